# baseline (speedup 1.0000x reference)
import functools

import numpy as np

import jax
import jax.numpy as jnp
from jax import lax
from jax.experimental import pallas as pl
from jax.experimental.pallas import tpu as pltpu
from jax.experimental.pallas import tpu_sc as plsc

D = 128
W = 144
G = 64
SB = 27
NC, NS = 2, 16
NW = NC * NS


def _prep_body(beta_ref, x_ref, xnb_ref, xpad_ref):
    x = x_ref[...]
    nrm = jnp.sqrt(jnp.sum(x * x, axis=1, keepdims=True))
    rn = 1.0 / jnp.maximum(nrm, 1e-12)
    xnb_ref[...] = x * rn * beta_ref[0]
    blk = x.shape[0]
    rn16 = jnp.broadcast_to(rn, (blk, 16))
    xpad_ref[...] = jnp.concatenate([x, rn16], axis=1)


def _prep(x, beta, blk=1000):
    n = x.shape[0]
    return pl.pallas_call(
        _prep_body,
        grid=(n // blk,),
        in_specs=[
            pl.BlockSpec((1,), lambda i: (jnp.int32(0),), memory_space=pltpu.SMEM),
            pl.BlockSpec((blk, D), lambda i: (i, jnp.int32(0))),
        ],
        out_specs=[
            pl.BlockSpec((blk, D), lambda i: (i, jnp.int32(0))),
            pl.BlockSpec((blk, W), lambda i: (i, jnp.int32(0))),
        ],
        out_shape=[
            jax.ShapeDtypeStruct((n, D), jnp.float32),
            jax.ShapeDtypeStruct((n, W), jnp.float32),
        ],
    )(beta, x)


def _edge_body(nb, np_rows, n_edges, n_total,
               xnb_hbm, xpad_hbm, packed_hbm,
               out_hbm,
               acc_sh, pbuf, rbuf0, rbuf1, cbuf0, cbuf1,
               abuf0, abuf1, bbuf0, bbuf1,
               sa0, sb0, sa1, sb1, sc0, sc1):
    cid = jnp.int32(lax.axis_index("c"))
    sid = jnp.int32(lax.axis_index("s"))
    wid = sid * jnp.int32(NC) + cid
    rows_per_tile = np_rows // NS
    zero16 = jnp.zeros((16,), jnp.float32)
    lanes = lax.iota(jnp.int32, 16)
    rbuf = (rbuf0, rbuf1)
    cbuf = (cbuf0, cbuf1)
    abuf = (abuf0, abuf1)
    bbuf = (bbuf0, bbuf1)
    sema = (sa0, sa1)
    semb = (sb0, sb1)
    semc = (sc0, sc1)

    @pl.loop(jnp.int32(0), jnp.int32(G))
    def _zero_buf(i):
        for j in range(W // 16):
            bbuf0[i, pl.ds(16 * j, 16)] = zero16

    @pl.loop(jnp.int32(0), jnp.int32(rows_per_tile // G))
    def _zero_acc(m):
        pltpu.sync_copy(
            bbuf0,
            acc_sh.at[pl.ds(sid * jnp.int32(rows_per_tile) + m * jnp.int32(G), G)])

    plsc.subcore_barrier()

    pb_base = wid * jnp.int32(nb)

    def _stage_idx(t, s):
        @pl.when(t >= jnp.int32(2))
        def _drain_scatter():
            pltpu.make_async_copy(bbuf[s], acc_sh.at[rbuf[s]], semc[s]).wait()

        pr = lax.rem(t, jnp.int32(SB))
        for i in range(G // 16):
            rbuf[s][pl.ds(16 * i, 16)] = pbuf[pr, pl.ds(16 * i, 16)]
            cbuf[s][pl.ds(16 * i, 16)] = pbuf[pr, pl.ds(G + 16 * i, 16)]
        pltpu.async_copy(xnb_hbm.at[rbuf[s]], abuf[s], sema[s])
        pltpu.async_copy(xpad_hbm.at[cbuf[s]], bbuf[s], semb[s])

    def _load_super(t):
        pltpu.sync_copy(packed_hbm.at[pl.ds(pb_base + t, SB)], pbuf)

    def _process(s):
        pltpu.make_async_copy(xnb_hbm.at[rbuf[s]], abuf[s], sema[s]).wait()
        pltpu.make_async_copy(xpad_hbm.at[cbuf[s]], bbuf[s], semb[s]).wait()

        @pl.loop(jnp.int32(0), jnp.int32(G // 16))
        def _dot(g):
            e0 = g * jnp.int32(16)

            def _four_edges(e4, carry):
                for u in range(4):
                    e = e0 + e4 * jnp.int32(4) + jnp.int32(u)
                    bv = [bbuf[s][e, pl.ds(16 * j, 16)] for j in range(D // 16)]
                    a0 = abuf[s][e, pl.ds(0, 16)] * bv[0]
                    a1 = abuf[s][e, pl.ds(16, 16)] * bv[1]
                    for j in range(2, D // 16, 2):
                        a0 = a0 + abuf[s][e, pl.ds(16 * j, 16)] * bv[j]
                        a1 = a1 + abuf[s][e, pl.ds(16 * (j + 1), 16)] * bv[j + 1]
                    se = jnp.sum(a0 + a1)
                    q = jnp.exp(se * bbuf[s][e, pl.ds(D, 16)])
                    for j in range(D // 16):
                        bbuf[s][e, pl.ds(16 * j, 16)] = bv[j] * q
                    bbuf[s][e, pl.ds(D, 16)] = q
                return carry

            lax.fori_loop(jnp.int32(0), jnp.int32(4), _four_edges, jnp.int32(0))

        pltpu.async_copy(bbuf[s], acc_sh.at[rbuf[s]], semc[s], add=True)

    _load_super(jnp.int32(0))
    _stage_idx(jnp.int32(0), 0)

    @pl.loop(jnp.int32(0), jnp.int32(nb // 2))
    def _pair(bp):
        for s in (0, 1):
            b = bp * jnp.int32(2) + jnp.int32(s)
            t = b + jnp.int32(1)

            @pl.when(t < jnp.int32(nb))
            def _prefetch():
                @pl.when(lax.rem(t, jnp.int32(SB)) == jnp.int32(0))
                def _reload():
                    _load_super(t)

                _stage_idx(t, 1 - s)

            _process(s)

    pltpu.make_async_copy(bbuf[0], acc_sh.at[rbuf[0]], semc[0]).wait()
    pltpu.make_async_copy(bbuf[1], acc_sh.at[rbuf[1]], semc[1]).wait()
    plsc.subcore_barrier()

    base_r = sid * jnp.int32(rows_per_tile)
    pltpu.sync_copy(acc_sh.at[pl.ds(base_r, rows_per_tile)],
                    out_hbm.at[cid, pl.ds(base_r, rows_per_tile)])


def _edge_pass(xnb, xpad, packed, np_rows, n_edges, n_total):
    nbtot = packed.shape[0]
    nb = nbtot // NW
    mesh = plsc.VectorSubcoreMesh(core_axis_name="c", subcore_axis_name="s",
                                  num_cores=NC, num_subcores=NS)
    body = functools.partial(_edge_body, nb, np_rows, n_edges, n_total)
    return pl.kernel(
        body,
        out_type=jax.ShapeDtypeStruct((NC, np_rows, W), jnp.float32),
        mesh=mesh,
        compiler_params=pltpu.CompilerParams(needs_layout_passes=False,
                                             use_tc_tiling_on_sc=False),
        scratch_types=[
            pltpu.VMEM_SHARED((np_rows, W), jnp.float32),
            pltpu.VMEM((SB, 2 * G), jnp.int32),
            pltpu.VMEM((G,), jnp.int32),
            pltpu.VMEM((G,), jnp.int32),
            pltpu.VMEM((G,), jnp.int32),
            pltpu.VMEM((G,), jnp.int32),
            pltpu.VMEM((G, D), jnp.float32),
            pltpu.VMEM((G, D), jnp.float32),
            pltpu.VMEM((G, W), jnp.float32),
            pltpu.VMEM((G, W), jnp.float32),
            pltpu.SemaphoreType.DMA,
            pltpu.SemaphoreType.DMA,
            pltpu.SemaphoreType.DMA,
            pltpu.SemaphoreType.DMA,
            pltpu.SemaphoreType.DMA,
            pltpu.SemaphoreType.DMA,
        ],
    )(xnb, xpad, packed)


def _combine_body(acc_ref, out_ref):
    a = acc_ref[0] + acc_ref[1]
    den = a[:, D:D + 1]
    out_ref[...] = a[:, :D] / (den + 1e-16)


def _combine(acc, n, blk=1000):
    return pl.pallas_call(
        _combine_body,
        grid=(n // blk,),
        in_specs=[pl.BlockSpec((NC, blk, W),
                               lambda i: (jnp.int32(0), i, jnp.int32(0)))],
        out_specs=pl.BlockSpec((blk, D), lambda i: (i, jnp.int32(0))),
        out_shape=jax.ShapeDtypeStruct((n, D), jnp.float32),
    )(acc)


def kernel(x, edge_index, beta):
    n, _ = x.shape
    e = edge_index.shape[1]
    et = e + n

    row0 = edge_index[0].astype(jnp.int32)
    col0 = edge_index[1].astype(jnp.int32)
    loop_idx = jnp.arange(n, dtype=jnp.int32)
    trash = jnp.int32(n)
    rows = jnp.concatenate([jnp.where(row0 != col0, row0, trash), loop_idx])
    cols = jnp.concatenate([col0, loop_idx])

    unit = NW * G
    nb = -(-et // unit)
    step = 2 * SB
    nb = ((nb + step - 1) // step) * step
    et_pad = nb * unit
    pad = et_pad - et
    rows = jnp.pad(rows, (0, pad), constant_values=n)
    cols = jnp.pad(cols, (0, pad))
    packed = jnp.concatenate(
        [rows.reshape(-1, G), cols.reshape(-1, G)], axis=1)

    np_rows = ((n + NS * G) // (NS * G)) * (NS * G)

    xnb, xpad = _prep(x, beta.astype(jnp.float32))
    acc = _edge_pass(xnb, xpad, packed, np_rows, e, et)
    return _combine(acc, n)


# speedup vs baseline: 1.3398x; 1.3398x over previous
import functools

import numpy as np

import jax
import jax.numpy as jnp
from jax import lax
from jax.experimental import pallas as pl
from jax.experimental.pallas import tpu as pltpu
from jax.experimental.pallas import tpu_sc as plsc

D = 128
W = 144
G = 64
SB = 27
NC, NS = 2, 16
NW = NC * NS


def _prep_body(beta_ref, x_ref, xnb_ref, xpad_ref):
    x = x_ref[...]
    nrm = jnp.sqrt(jnp.sum(x * x, axis=1, keepdims=True))
    rn = 1.0 / jnp.maximum(nrm, 1e-12)
    xnb_ref[...] = x * rn * beta_ref[0]
    blk = x.shape[0]
    rn16 = jnp.broadcast_to(rn, (blk, 16))
    xpad_ref[...] = jnp.concatenate([x, rn16], axis=1)


def _prep(x, beta, blk=1000):
    n = x.shape[0]
    return pl.pallas_call(
        _prep_body,
        grid=(n // blk,),
        in_specs=[
            pl.BlockSpec((1,), lambda i: (jnp.int32(0),), memory_space=pltpu.SMEM),
            pl.BlockSpec((blk, D), lambda i: (i, jnp.int32(0))),
        ],
        out_specs=[
            pl.BlockSpec((blk, D), lambda i: (i, jnp.int32(0))),
            pl.BlockSpec((blk, W), lambda i: (i, jnp.int32(0))),
        ],
        out_shape=[
            jax.ShapeDtypeStruct((n, D), jnp.float32),
            jax.ShapeDtypeStruct((n, W), jnp.float32),
        ],
    )(beta, x)


def _edge_body(nb, np_rows, n_edges, n_total,
               xnb_hbm, xpad_hbm, packed_hbm,
               out_hbm,
               acc_sh, pbuf, rbuf0, rbuf1, cbuf0, cbuf1,
               abuf0, abuf1, bbuf0, bbuf1, sbuf,
               sa0, sb0, sa1, sb1, sc0, sc1):
    cid = jnp.int32(lax.axis_index("c"))
    sid = jnp.int32(lax.axis_index("s"))
    wid = sid * jnp.int32(NC) + cid
    rows_per_tile = np_rows // NS
    zero16 = jnp.zeros((16,), jnp.float32)
    lanes = lax.iota(jnp.int32, 16)
    rbuf = (rbuf0, rbuf1)
    cbuf = (cbuf0, cbuf1)
    abuf = (abuf0, abuf1)
    bbuf = (bbuf0, bbuf1)
    sema = (sa0, sa1)
    semb = (sb0, sb1)
    semc = (sc0, sc1)

    @pl.loop(jnp.int32(0), jnp.int32(G))
    def _zero_buf(i):
        for j in range(W // 16):
            bbuf0[i, pl.ds(16 * j, 16)] = zero16

    @pl.loop(jnp.int32(0), jnp.int32(rows_per_tile // G))
    def _zero_acc(m):
        pltpu.async_copy(
            bbuf0,
            acc_sh.at[pl.ds(sid * jnp.int32(rows_per_tile) + m * jnp.int32(G), G)],
            sc0)

    @pl.loop(jnp.int32(0), jnp.int32(rows_per_tile // G))
    def _zero_wait(m):
        pltpu.make_async_copy(
            bbuf0,
            acc_sh.at[pl.ds(sid * jnp.int32(rows_per_tile) + m * jnp.int32(G), G)],
            sc0).wait()

    plsc.subcore_barrier()

    pb_base = wid * jnp.int32(nb)

    def _stage_idx(t, s):
        @pl.when(t >= jnp.int32(2))
        def _drain_scatter():
            pltpu.make_async_copy(bbuf[s], acc_sh.at[rbuf[s]], semc[s]).wait()

        pr = lax.rem(t, jnp.int32(SB))
        for i in range(G // 16):
            rbuf[s][pl.ds(16 * i, 16)] = pbuf[pr, pl.ds(16 * i, 16)]
            cbuf[s][pl.ds(16 * i, 16)] = pbuf[pr, pl.ds(G + 16 * i, 16)]
        pltpu.async_copy(xnb_hbm.at[rbuf[s]], abuf[s], sema[s])
        pltpu.async_copy(xpad_hbm.at[cbuf[s]], bbuf[s], semb[s])

    def _load_super(t):
        pltpu.sync_copy(packed_hbm.at[pl.ds(pb_base + t, SB)], pbuf)

    def _process(b, s):
        pltpu.make_async_copy(xnb_hbm.at[rbuf[s]], abuf[s], sema[s]).wait()
        pltpu.make_async_copy(xpad_hbm.at[cbuf[s]], bbuf[s], semb[s]).wait()
        e_base = wid * jnp.int32(nb * G) + b * jnp.int32(G)

        @pl.loop(jnp.int32(0), jnp.int32(G // 16))
        def _dot(g):
            e0 = g * jnp.int32(16)

            def _four_edges(e4, sv):
                for u in range(4):
                    e2 = e4 * jnp.int32(4) + jnp.int32(u)
                    e = e0 + e2
                    a0 = abuf[s][e, pl.ds(0, 16)] * bbuf[s][e, pl.ds(0, 16)]
                    a1 = abuf[s][e, pl.ds(16, 16)] * bbuf[s][e, pl.ds(16, 16)]
                    for j in range(2, D // 16, 2):
                        a0 = a0 + abuf[s][e, pl.ds(16 * j, 16)] * bbuf[s][e, pl.ds(16 * j, 16)]
                        a1 = a1 + abuf[s][e, pl.ds(16 * (j + 1), 16)] * bbuf[s][e, pl.ds(16 * (j + 1), 16)]
                    se = jnp.sum(a0 + a1)
                    sv = jnp.where(lanes == e2, se * bbuf[s][e, pl.ds(D, 16)], sv)
                return sv

            sv = lax.fori_loop(jnp.int32(0), jnp.int32(4), _four_edges,
                               jnp.zeros((16,), jnp.float32))
            rv = rbuf[s][pl.ds(e0, 16)]
            cv = cbuf[s][pl.ds(e0, 16)]
            ev = e_base + e0 + lanes
            ok = (rv != cv) | ((ev >= jnp.int32(n_edges)) & (ev < jnp.int32(n_total)))
            sbuf[pl.ds(e0, 16)] = jnp.where(ok, jnp.exp(sv), 0.0)

        @pl.loop(jnp.int32(0), jnp.int32(G // 16))
        def _scale(g):
            e0 = g * jnp.int32(16)
            pv = sbuf[pl.ds(e0, 16)]
            for e2 in range(16):
                e = e0 + e2
                q = pv[e2]
                for j in range(D // 16):
                    bbuf[s][e, pl.ds(16 * j, 16)] = bbuf[s][e, pl.ds(16 * j, 16)] * q
                bbuf[s][e, pl.ds(D, 16)] = jnp.full((16,), q, jnp.float32)

        pltpu.async_copy(bbuf[s], acc_sh.at[rbuf[s]], semc[s], add=True)

    _load_super(jnp.int32(0))
    _stage_idx(jnp.int32(0), 0)

    @pl.loop(jnp.int32(0), jnp.int32(nb // 2))
    def _pair(bp):
        for s in (0, 1):
            b = bp * jnp.int32(2) + jnp.int32(s)
            t = b + jnp.int32(1)

            @pl.when(t < jnp.int32(nb))
            def _prefetch():
                @pl.when(lax.rem(t, jnp.int32(SB)) == jnp.int32(0))
                def _reload():
                    _load_super(t)

                _stage_idx(t, 1 - s)

            _process(b, s)

    pltpu.make_async_copy(bbuf[0], acc_sh.at[rbuf[0]], semc[0]).wait()
    pltpu.make_async_copy(bbuf[1], acc_sh.at[rbuf[1]], semc[1]).wait()
    plsc.subcore_barrier()

    base_r = sid * jnp.int32(rows_per_tile)
    pltpu.sync_copy(acc_sh.at[pl.ds(base_r, rows_per_tile)],
                    out_hbm.at[cid, pl.ds(base_r, rows_per_tile)])


def _edge_pass(xnb, xpad, packed, np_rows, n_edges, n_total):
    nbtot = packed.shape[0]
    nb = nbtot // NW
    mesh = plsc.VectorSubcoreMesh(core_axis_name="c", subcore_axis_name="s",
                                  num_cores=NC, num_subcores=NS)
    body = functools.partial(_edge_body, nb, np_rows, n_edges, n_total)
    return pl.kernel(
        body,
        out_type=jax.ShapeDtypeStruct((NC, np_rows, W), jnp.float32),
        mesh=mesh,
        compiler_params=pltpu.CompilerParams(needs_layout_passes=False,
                                             use_tc_tiling_on_sc=False),
        scratch_types=[
            pltpu.VMEM_SHARED((np_rows, W), jnp.float32),
            pltpu.VMEM((SB, 2 * G), jnp.int32),
            pltpu.VMEM((G,), jnp.int32),
            pltpu.VMEM((G,), jnp.int32),
            pltpu.VMEM((G,), jnp.int32),
            pltpu.VMEM((G,), jnp.int32),
            pltpu.VMEM((G, D), jnp.float32),
            pltpu.VMEM((G, D), jnp.float32),
            pltpu.VMEM((G, W), jnp.float32),
            pltpu.VMEM((G, W), jnp.float32),
            pltpu.VMEM((G,), jnp.float32),
            pltpu.SemaphoreType.DMA,
            pltpu.SemaphoreType.DMA,
            pltpu.SemaphoreType.DMA,
            pltpu.SemaphoreType.DMA,
            pltpu.SemaphoreType.DMA,
            pltpu.SemaphoreType.DMA,
        ],
    )(xnb, xpad, packed)


def _combine_body(acc_ref, out_ref):
    a = acc_ref[0] + acc_ref[1]
    den = a[:, D:D + 1]
    out_ref[...] = a[:, :D] / (den + 1e-16)


def _combine(acc, n, blk=1000):
    return pl.pallas_call(
        _combine_body,
        grid=(n // blk,),
        in_specs=[pl.BlockSpec((NC, blk, W),
                               lambda i: (jnp.int32(0), i, jnp.int32(0)))],
        out_specs=pl.BlockSpec((blk, D), lambda i: (i, jnp.int32(0))),
        out_shape=jax.ShapeDtypeStruct((n, D), jnp.float32),
    )(acc)


def kernel(x, edge_index, beta):
    n, _ = x.shape
    e = edge_index.shape[1]
    et = e + n

    row0 = edge_index[0].astype(jnp.int32)
    col0 = edge_index[1].astype(jnp.int32)
    loop_idx = jnp.arange(n, dtype=jnp.int32)
    rows = jnp.concatenate([row0, loop_idx])
    cols = jnp.concatenate([col0, loop_idx])

    unit = NW * G
    nb = -(-et // unit)
    step = 2 * SB
    nb = ((nb + step - 1) // step) * step
    et_pad = nb * unit
    pad = et_pad - et
    rows = jnp.pad(rows, (0, pad))
    cols = jnp.pad(cols, (0, pad))
    packed = jnp.concatenate(
        [rows.reshape(-1, G), cols.reshape(-1, G)], axis=1)

    np_rows = ((n + NS * G - 1) // (NS * G)) * (NS * G)

    xnb, xpad = _prep(x, beta.astype(jnp.float32))
    acc = _edge_pass(xnb, xpad, packed, np_rows, e, et)
    return _combine(acc, n)


# speedup vs baseline: 1.3404x; 1.0004x over previous
import functools

import jax
import jax.numpy as jnp
from jax import lax
from jax.experimental import pallas as pl
from jax.experimental.pallas import tpu as pltpu
from jax.experimental.pallas import tpu_sc as plsc

D = 128
W = 144
G = 64
SB = 27
NC, NS = 2, 16
NW = NC * NS


def _prep_body(beta_ref, x_ref, xnb_ref, xpad_ref):
    x = x_ref[...]
    nrm = jnp.sqrt(jnp.sum(x * x, axis=1, keepdims=True))
    rn = 1.0 / jnp.maximum(nrm, 1e-12)
    xnb_ref[...] = x * rn * beta_ref[0]
    blk = x.shape[0]
    rn16 = jnp.broadcast_to(rn, (blk, 16))
    xpad_ref[...] = jnp.concatenate([x, rn16], axis=1)


def _prep(x, beta, blk=1000):
    n = x.shape[0]
    return pl.pallas_call(
        _prep_body,
        grid=(n // blk,),
        in_specs=[
            pl.BlockSpec((1,), lambda i: (jnp.int32(0),), memory_space=pltpu.SMEM),
            pl.BlockSpec((blk, D), lambda i: (i, jnp.int32(0))),
        ],
        out_specs=[
            pl.BlockSpec((blk, D), lambda i: (i, jnp.int32(0))),
            pl.BlockSpec((blk, W), lambda i: (i, jnp.int32(0))),
        ],
        out_shape=[
            jax.ShapeDtypeStruct((n, D), jnp.float32),
            jax.ShapeDtypeStruct((n, W), jnp.float32),
        ],
    )(beta, x)


def _edge_body(nb, np_rows, n_edges, n_total,
               xnb_hbm, xpad_hbm, packed_hbm,
               out_hbm,
               acc_sh, pbuf, rbuf0, rbuf1, cbuf0, cbuf1,
               abuf0, abuf1, bbuf0, bbuf1, sbuf,
               sa0, sb0, sa1, sb1, sc0, sc1):
    cid = jnp.int32(lax.axis_index("c"))
    sid = jnp.int32(lax.axis_index("s"))
    wid = sid * jnp.int32(NC) + cid
    rows_per_tile = np_rows // NS
    zero16 = jnp.zeros((16,), jnp.float32)
    lanes = lax.iota(jnp.int32, 16)
    rbuf = (rbuf0, rbuf1)
    cbuf = (cbuf0, cbuf1)
    abuf = (abuf0, abuf1)
    bbuf = (bbuf0, bbuf1)
    sema = (sa0, sa1)
    semb = (sb0, sb1)
    semc = (sc0, sc1)

    @pl.loop(jnp.int32(0), jnp.int32(G))
    def _zero_buf(i):
        for j in range(W // 16):
            bbuf0[i, pl.ds(16 * j, 16)] = zero16

    @pl.loop(jnp.int32(0), jnp.int32(rows_per_tile // G))
    def _zero_acc(m):
        pltpu.async_copy(
            bbuf0,
            acc_sh.at[pl.ds(sid * jnp.int32(rows_per_tile) + m * jnp.int32(G), G)],
            sc0)

    @pl.loop(jnp.int32(0), jnp.int32(rows_per_tile // G))
    def _zero_wait(m):
        pltpu.make_async_copy(
            bbuf0,
            acc_sh.at[pl.ds(sid * jnp.int32(rows_per_tile) + m * jnp.int32(G), G)],
            sc0).wait()

    plsc.subcore_barrier()

    pb_base = wid * jnp.int32(nb)

    def _stage_idx(t, s):
        @pl.when(t >= jnp.int32(2))
        def _drain_scatter():
            pltpu.make_async_copy(bbuf[s], acc_sh.at[rbuf[s]], semc[s]).wait()

        pr = lax.rem(t, jnp.int32(SB))
        for i in range(G // 16):
            rbuf[s][pl.ds(16 * i, 16)] = pbuf[pr, pl.ds(16 * i, 16)]
            cbuf[s][pl.ds(16 * i, 16)] = pbuf[pr, pl.ds(G + 16 * i, 16)]
        pltpu.async_copy(xnb_hbm.at[rbuf[s]], abuf[s], sema[s])
        pltpu.async_copy(xpad_hbm.at[cbuf[s]], bbuf[s], semb[s])

    def _load_super(t):
        pltpu.sync_copy(packed_hbm.at[pl.ds(pb_base + t, SB)], pbuf)

    def _process(b, s):
        pltpu.make_async_copy(xnb_hbm.at[rbuf[s]], abuf[s], sema[s]).wait()
        pltpu.make_async_copy(xpad_hbm.at[cbuf[s]], bbuf[s], semb[s]).wait()
        e_base = wid * jnp.int32(nb * G) + b * jnp.int32(G)

        @pl.loop(jnp.int32(0), jnp.int32(G // 16))
        def _dot(g):
            e0 = g * jnp.int32(16)

            def _four_edges(e4, sv):
                for u in range(4):
                    e2 = e4 * jnp.int32(4) + jnp.int32(u)
                    e = e0 + e2
                    a0 = abuf[s][e, pl.ds(0, 16)] * bbuf[s][e, pl.ds(0, 16)]
                    a1 = abuf[s][e, pl.ds(16, 16)] * bbuf[s][e, pl.ds(16, 16)]
                    for j in range(2, D // 16, 2):
                        a0 = a0 + abuf[s][e, pl.ds(16 * j, 16)] * bbuf[s][e, pl.ds(16 * j, 16)]
                        a1 = a1 + abuf[s][e, pl.ds(16 * (j + 1), 16)] * bbuf[s][e, pl.ds(16 * (j + 1), 16)]
                    se = jnp.sum(a0 + a1)
                    sv = jnp.where(lanes == e2, se * bbuf[s][e, pl.ds(D, 16)], sv)
                return sv

            sv = lax.fori_loop(jnp.int32(0), jnp.int32(4), _four_edges,
                               jnp.zeros((16,), jnp.float32))
            rv = rbuf[s][pl.ds(e0, 16)]
            cv = cbuf[s][pl.ds(e0, 16)]
            ev = e_base + e0 + lanes
            ok = (rv != cv) | ((ev >= jnp.int32(n_edges)) & (ev < jnp.int32(n_total)))
            sbuf[pl.ds(e0, 16)] = jnp.where(ok, jnp.exp(sv), 0.0)

        @pl.loop(jnp.int32(0), jnp.int32(G // 16))
        def _scale(g):
            e0 = g * jnp.int32(16)
            pv = sbuf[pl.ds(e0, 16)]
            for e2 in range(16):
                e = e0 + e2
                q = pv[e2]
                for j in range(D // 16):
                    bbuf[s][e, pl.ds(16 * j, 16)] = bbuf[s][e, pl.ds(16 * j, 16)] * q
                bbuf[s][e, pl.ds(D, 16)] = jnp.full((16,), q, jnp.float32)

        pltpu.async_copy(bbuf[s], acc_sh.at[rbuf[s]], semc[s], add=True)

    _load_super(jnp.int32(0))
    _stage_idx(jnp.int32(0), 0)

    @pl.loop(jnp.int32(0), jnp.int32(nb // 2))
    def _pair(bp):
        for s in (0, 1):
            b = bp * jnp.int32(2) + jnp.int32(s)
            t = b + jnp.int32(1)

            @pl.when(t < jnp.int32(nb))
            def _prefetch():
                @pl.when(lax.rem(t, jnp.int32(SB)) == jnp.int32(0))
                def _reload():
                    _load_super(t)

                _stage_idx(t, 1 - s)

            _process(b, s)

    pltpu.make_async_copy(bbuf[0], acc_sh.at[rbuf[0]], semc[0]).wait()
    pltpu.make_async_copy(bbuf[1], acc_sh.at[rbuf[1]], semc[1]).wait()
    plsc.subcore_barrier()

    base_r = sid * jnp.int32(rows_per_tile)
    pltpu.sync_copy(acc_sh.at[pl.ds(base_r, rows_per_tile)],
                    out_hbm.at[cid, pl.ds(base_r, rows_per_tile)])


def _edge_pass(xnb, xpad, packed, np_rows, n_edges, n_total):
    nbtot = packed.shape[0]
    nb = nbtot // NW
    mesh = plsc.VectorSubcoreMesh(core_axis_name="c", subcore_axis_name="s",
                                  num_cores=NC, num_subcores=NS)
    body = functools.partial(_edge_body, nb, np_rows, n_edges, n_total)
    return pl.kernel(
        body,
        out_type=jax.ShapeDtypeStruct((NC, np_rows, W), jnp.float32),
        mesh=mesh,
        compiler_params=pltpu.CompilerParams(needs_layout_passes=False,
                                             use_tc_tiling_on_sc=False),
        scratch_types=[
            pltpu.VMEM_SHARED((np_rows, W), jnp.float32),
            pltpu.VMEM((SB, 2 * G), jnp.int32),
            pltpu.VMEM((G,), jnp.int32),
            pltpu.VMEM((G,), jnp.int32),
            pltpu.VMEM((G,), jnp.int32),
            pltpu.VMEM((G,), jnp.int32),
            pltpu.VMEM((G, D), jnp.float32),
            pltpu.VMEM((G, D), jnp.float32),
            pltpu.VMEM((G, W), jnp.float32),
            pltpu.VMEM((G, W), jnp.float32),
            pltpu.VMEM((G,), jnp.float32),
            pltpu.SemaphoreType.DMA,
            pltpu.SemaphoreType.DMA,
            pltpu.SemaphoreType.DMA,
            pltpu.SemaphoreType.DMA,
            pltpu.SemaphoreType.DMA,
            pltpu.SemaphoreType.DMA,
        ],
    )(xnb, xpad, packed)


def _combine_body(acc_ref, out_ref):
    a = acc_ref[0] + acc_ref[1]
    den = a[:, D:D + 1]
    out_ref[...] = a[:, :D] / (den + 1e-16)


def _combine(acc, n, blk=1000):
    return pl.pallas_call(
        _combine_body,
        grid=(n // blk,),
        in_specs=[pl.BlockSpec((NC, blk, W),
                               lambda i: (jnp.int32(0), i, jnp.int32(0)))],
        out_specs=pl.BlockSpec((blk, D), lambda i: (i, jnp.int32(0))),
        out_shape=jax.ShapeDtypeStruct((n, D), jnp.float32),
    )(acc)


def kernel(x, edge_index, beta):
    n, _ = x.shape
    e = edge_index.shape[1]
    et = e + n

    row0 = edge_index[0].astype(jnp.int32)
    col0 = edge_index[1].astype(jnp.int32)
    loop_idx = jnp.arange(n, dtype=jnp.int32)
    rows = jnp.concatenate([row0, loop_idx])
    cols = jnp.concatenate([col0, loop_idx])

    unit = NW * G
    nb = -(-et // unit)
    step = 2 * SB
    nb = ((nb + step - 1) // step) * step
    et_pad = nb * unit
    pad = et_pad - et
    rows = jnp.pad(rows, (0, pad))
    cols = jnp.pad(cols, (0, pad))
    packed = jnp.concatenate(
        [rows.reshape(-1, G), cols.reshape(-1, G)], axis=1)

    np_rows = ((n + NS * G - 1) // (NS * G)) * (NS * G)

    xnb, xpad = _prep(x, beta.astype(jnp.float32))
    acc = _edge_pass(xnb, xpad, packed, np_rows, e, et)
    return _combine(acc, n)
